# baseline (speedup 1.0000x reference)
import functools

import jax
import jax.numpy as jnp
from jax import lax
from jax.experimental import pallas as pl
from jax.experimental.pallas import tpu as pltpu
from jax.experimental.pallas import tpu_sc as plsc

S = 2048
SENT = 4 * S * S
_A = 6378137.0
_E2 = 6.69437999014e-3

_LANES = 1024
_RBLK = 8

NT = 32
NB = 512
BSZ = SENT // NB
VSHIFT = 15
NPAD = 1007616
PPT = NPAD // NT
CH = 3936
NCH = PPT // CH
CH3 = 4096
CNTW = 8320
LROWS = 2482
MG = 1273888
GROWS = MG // 16
PIX_SENT = 32768


def _transform_body(scal_ref, pts_ref, key_ref, int_ref, *, n_valid, rows_per_blk):
    lat = pts_ref[0]
    lon = pts_ref[1]
    alt = pts_ref[2]

    x0 = scal_ref[0]
    y0 = scal_ref[1]
    z0 = scal_ref[2]
    neg_so = scal_ref[3]
    co = scal_ref[4]
    neg_sl_co = scal_ref[5]
    sl_so = scal_ref[6]
    cl = scal_ref[7]
    cl_co = scal_ref[8]
    cl_so = scal_ref[9]
    sl = scal_ref[10]
    r00 = scal_ref[11]
    r01 = scal_ref[12]
    r02 = scal_ref[13]
    r10 = scal_ref[14]
    r11 = scal_ref[15]
    r12 = scal_ref[16]
    r20 = scal_ref[17]
    r21 = scal_ref[18]
    r22 = scal_ref[19]

    latr = jnp.deg2rad(lat)
    lonr = jnp.deg2rad(lon)
    s = jnp.sin(latr)
    c = jnp.cos(latr)
    Nv = _A * lax.rsqrt(1.0 - _E2 * s * s)
    X = (Nv + alt) * c * jnp.cos(lonr)
    Y = (Nv + alt) * c * jnp.sin(lonr)
    Z = (Nv * (1.0 - _E2) + alt) * s

    dx = X - x0
    dy = Y - y0
    dz = Z - z0
    e = neg_so * dx + co * dy
    n = neg_sl_co * dx - sl_so * dy + cl * dz
    u = cl_co * dx + cl_so * dy + sl * dz

    x = r00 * e + r01 * n + r02 * u
    y = r10 * e + r11 * n + r12 * u
    z = r20 * e + r21 * n + r22 * u

    ax = jnp.abs(x)
    ay = jnp.abs(y)
    az = jnp.abs(z)
    m_front = (z > 0) & (z > ax) & (z > ay)
    m_back = (z < 0) & (-z > ax) & (-z > ay)
    m_right = (x > 0) & (x > az) & (x > ay)
    m_left = (x < 0) & (-x > az) & (-x > ay)

    f = S / 2.0

    def cam2key(px_x, px_y, px_z):
        z_safe = jnp.where(jnp.abs(px_z) > 1e-9, px_z, 1.0)
        pu = f * px_x / z_safe + f
        pv = f * px_y / z_safe + f
        iu = jnp.clip(jnp.floor(pu), 0, S - 1).astype(jnp.int32)
        iv = jnp.clip(jnp.floor(pv), 0, S - 1).astype(jnp.int32)
        return iu * S + iv

    kf = cam2key(x, y, z)
    kb = cam2key(x, -y, z)
    kr = cam2key(-z, y, x)
    kl = cam2key(z, y, -x)

    key = jnp.where(
        m_front, kf,
        jnp.where(m_back, S * S + kb,
                  jnp.where(m_left, 2 * S * S + kl,
                            jnp.where(m_right, 3 * S * S + kr, SENT))))

    pid = pl.program_id(0)
    row = lax.broadcasted_iota(jnp.int32, (rows_per_blk, _LANES), 0)
    col = lax.broadcasted_iota(jnp.int32, (rows_per_blk, _LANES), 1)
    gidx = (pid * rows_per_blk + row) * _LANES + col
    key_ref[...] = jnp.where(gidx < n_valid, key, SENT)
    int_ref[...] = pts_ref[3]


def _compute_keys(points, cam_params):
    n = points.shape[0]
    rows = NPAD // _LANES
    grid = rows // _RBLK

    pts_t = jnp.transpose(points)
    pts_t = jnp.pad(pts_t, ((0, 0), (0, NPAD - n)))
    pts_t = pts_t.reshape(4, rows, _LANES)

    lat0, lon0, alt0 = cam_params[0], cam_params[1], cam_params[2]
    latr0 = jnp.deg2rad(lat0)
    lonr0 = jnp.deg2rad(lon0)
    sl = jnp.sin(latr0)
    cl = jnp.cos(latr0)
    so = jnp.sin(lonr0)
    co = jnp.cos(lonr0)
    Nv0 = _A / jnp.sqrt(1.0 - _E2 * sl * sl)
    x0 = (Nv0 + alt0) * cl * jnp.cos(lonr0)
    y0 = (Nv0 + alt0) * cl * jnp.sin(lonr0)
    z0 = (Nv0 * (1.0 - _E2) + alt0) * sl

    qs = -cam_params[3]
    qx = cam_params[4]
    qy = cam_params[5]
    qz = cam_params[6]
    nrm = jnp.sqrt(qs * qs + qx * qx + qy * qy + qz * qz) + 1e-12
    qs, qx, qy, qz = qs / nrm, qx / nrm, qy / nrm, qz / nrm
    r00 = 1 - 2 * (qy * qy + qz * qz)
    r01 = 2 * (qx * qy - qz * qs)
    r02 = 2 * (qx * qz + qy * qs)
    r10 = 2 * (qx * qy + qz * qs)
    r11 = 1 - 2 * (qx * qx + qz * qz)
    r12 = 2 * (qy * qz - qx * qs)
    r20 = 2 * (qx * qz - qy * qs)
    r21 = 2 * (qy * qz + qx * qs)
    r22 = 1 - 2 * (qx * qx + qy * qy)

    scal = jnp.stack([
        x0, y0, z0, -so, co, -sl * co, sl * so, cl, cl * co, cl * so, sl,
        r00, r01, r02, r10, r11, r12, r20, r21, r22,
    ]).astype(jnp.float32)

    keys, inten = pl.pallas_call(
        functools.partial(_transform_body, n_valid=n, rows_per_blk=_RBLK),
        grid=(grid,),
        in_specs=[
            pl.BlockSpec(memory_space=pltpu.SMEM),
            pl.BlockSpec((4, _RBLK, _LANES), lambda i: (0, i, 0)),
        ],
        out_specs=[pl.BlockSpec((_RBLK, _LANES), lambda i: (i, 0)),
                   pl.BlockSpec((_RBLK, _LANES), lambda i: (i, 0))],
        out_shape=[jax.ShapeDtypeStruct((rows, _LANES), jnp.int32),
                   jax.ShapeDtypeStruct((rows, _LANES), jnp.float32)],
    )(scal, pts_t)
    return keys.reshape(-1), inten.reshape(-1)


def _wid():
    return lax.axis_index("s") * 2 + lax.axis_index("c")


_IOTA = lambda: lax.broadcasted_iota(jnp.int32, (16,), 0)


def _sc_count_body(keys_hbm, tb_hbm, cnt3_hbm, cnt3_v, tbrow_v, keybuf):
    w = _wid()
    iota = _IOTA()

    def zero_body(i, _):
        cnt3_v[pl.ds(i * 16, 16)] = jnp.zeros((16,), jnp.int32)
        return 0
    lax.fori_loop(0, CNTW // 16, zero_body, 0)

    def chunk_body(c, _):
        pltpu.sync_copy(keys_hbm.at[pl.ds(w * PPT + c * CH, CH)], keybuf)

        def vec_body(i, _):
            k = keybuf[pl.ds(i * 16, 16)]
            b = lax.shift_right_logical(k, VSHIFT)
            cidx = iota * 513 + b
            cv = plsc.load_gather(cnt3_v, [cidx])
            plsc.store_scatter(cnt3_v, [cidx], cv + 1)
            return 0
        lax.fori_loop(0, CH // 16, vec_body, 0)
        return 0
    lax.fori_loop(0, NCH, chunk_body, 0)

    def tb_body(bc, _):
        acc = jnp.zeros((16,), jnp.int32)
        for l in range(16):
            acc = acc + cnt3_v[pl.ds(l * 513 + bc * 16, 16)]
        tbrow_v[pl.ds(bc * 16, 16)] = acc
        return 0
    lax.fori_loop(0, NB // 16, tb_body, 0)

    pltpu.sync_copy(tbrow_v, tb_hbm.at[pl.ds(w * NB, NB)])
    pltpu.sync_copy(cnt3_v, cnt3_hbm.at[pl.ds(w * CNTW, CNTW)])


def _sc_place_body(keys_hbm, inten_hbm, tb_hbm,
                   bpix_hbm, bint_hbm,
                   tb_v, cnt1_v, pix_v, intl_v, rowb_v, rowidx_v, shift_v,
                   keybuf, intbuf, s16_v, sem):
    w = _wid()
    iota = _IOTA()
    pltpu.sync_copy(tb_hbm, tb_v)

    def zero_rowb(i, _):
        rowb_v[pl.ds(i * 16, 16)] = jnp.zeros((16,), jnp.int32)
        return 0
    lax.fori_loop(0, 2560 // 16, zero_rowb, 0)

    lane0 = _IOTA() == 0

    def base_body(b, carry):
        g_run, lb_run = carry
        tot = jnp.max(plsc.load_gather(tb_v, [jnp.full((16,), w * NB + b,
                                                       jnp.int32)]))
        plsc.store_scatter(cnt1_v, [jnp.full((16,), b, jnp.int32)],
                           jnp.full((16,), lb_run, jnp.int32), mask=lane0)
        c1 = plsc.load_gather(tb_v, [iota * NB + b])
        c2 = plsc.load_gather(tb_v, [(iota + 16) * NB + b])
        p1 = (c1 + 15) & -16
        p2 = (c2 + 15) & -16
        zero = jnp.zeros((16,), jnp.int32)
        cross = (jnp.sum(jnp.where(iota < w, p1, zero)) +
                 jnp.sum(jnp.where(iota < w - 16, p2, zero)))
        rstart = lax.shift_right_logical(lb_run, 4)
        plsc.store_scatter(rowb_v, [jnp.full((16,), rstart, jnp.int32)],
                           jnp.full((16,), b, jnp.int32), mask=lane0)
        shift = lax.shift_right_logical((g_run + cross) - lb_run, 4)
        plsc.store_scatter(shift_v, [jnp.full((16,), b, jnp.int32)],
                           jnp.full((16,), shift, jnp.int32), mask=lane0)
        g_next = g_run + (jnp.sum(p1) + jnp.sum(p2))
        lb_next = lb_run + ((tot + 15) & -16)
        return (g_next, lb_next)
    _, lb_total = lax.fori_loop(0, NB, base_body,
                                (jnp.int32(0), jnp.int32(0)))

    def fill_body(r, _):
        plsc.store_scatter(pix_v, [jnp.full((16,), r, jnp.int32), iota],
                           jnp.full((16,), PIX_SENT, jnp.int32))
        return 0
    lax.fori_loop(0, LROWS, fill_body, 0)

    def chunk_body(c, _):
        off = w * PPT + c * CH
        pltpu.sync_copy(keys_hbm.at[pl.ds(off, CH)], keybuf)
        pltpu.sync_copy(inten_hbm.at[pl.ds(off, CH)], intbuf)

        def vec_body(i, _):
            k = keybuf[pl.ds(i * 16, 16)]
            b = lax.shift_right_logical(k, VSHIFT)
            valid = b < NB
            ck = b * 16 + iota
            sk, perm = plsc.sort_key_val(ck, iota)
            bs = lax.shift_right_logical(sk, 4)
            prev = bs.at[jnp.maximum(iota - 1, 0)].get(
                mode="promise_in_bounds")
            nxt = bs.at[jnp.minimum(iota + 1, 15)].get(
                mode="promise_in_bounds")
            zero = jnp.zeros((16,), jnp.int32)
            neq0 = (iota == 0) | (bs != prev)
            sp = jnp.where(neq0, iota, zero)
            for d in (1, 2, 4, 8):
                shd = sp.at[jnp.maximum(iota - d, 0)].get(
                    mode="promise_in_bounds")
                sp = jnp.maximum(sp, jnp.where(iota >= d, shd, zero))
            crank = iota - sp
            cv = plsc.load_gather(cnt1_v, [bs])
            slot_sorted = cv + crank
            islast = (iota == 15) | (bs != nxt)
            plsc.store_scatter(cnt1_v, [bs], slot_sorted + 1, mask=islast)
            pix_s = (k & (BSZ - 1)).at[perm].get(mode="promise_in_bounds")
            int_s = intbuf[pl.ds(i * 16, 16)].at[perm].get(
                mode="promise_in_bounds")
            valid_s = sk < NB * 16
            r = jnp.where(valid_s, lax.shift_right_logical(slot_sorted, 4), 0)
            col = jnp.where(valid_s, slot_sorted & 15, 0)
            plsc.store_scatter(pix_v, [r, col], pix_s, mask=valid_s)
            plsc.store_scatter(intl_v, [r, col], int_s, mask=valid_s)
            return 0
        lax.fori_loop(0, CH // 16, vec_body, 0)
        return 0
    lax.fori_loop(0, NCH, chunk_body, 0)

    used_rows = lax.shift_right_logical(lb_total, 4)

    def row_body(i, carry):
        rb = rowb_v[pl.ds(i * 16, 16)]
        cm = jnp.maximum(plsc.cummax(rb), jnp.full((16,), carry, jnp.int32))
        sh = plsc.load_gather(shift_v, [cm])
        lrow = i * 16 + iota
        gr = jnp.where(lrow < used_rows, lrow + sh, GROWS - 1)
        rowidx_v[pl.ds(i * 16, 16)] = gr
        return jnp.max(cm)
    lax.fori_loop(0, (LROWS + 15) // 16, row_body, jnp.int32(0))

    pltpu.async_copy(pix_v, bpix_hbm.at[rowidx_v], sem).wait()
    pltpu.async_copy(intl_v, bint_hbm.at[rowidx_v], sem).wait()


def _sc_emit_body(bpix_hbm, bint_hbm, tb_hbm, out_hbm,
                  tb_v, idxreg, valreg, pixbuf, intbuf,
                  start_smem, cnt_smem):
    w = _wid()
    iota = _IOTA()
    pltpu.sync_copy(tb_hbm, tb_v)

    def pref_body(b, g_run):
        c1 = plsc.load_gather(tb_v, [iota * NB + b])
        c2 = plsc.load_gather(tb_v, [(iota + 16) * NB + b])
        p1 = (c1 + 15) & -16
        p2 = (c2 + 15) & -16
        padded = jnp.sum(p1) + jnp.sum(p2)
        j = lax.shift_right_logical(b, 5)

        @pl.when((b & 31) == w)
        def _():
            start_smem[j] = g_run
            cnt_smem[j] = padded
        return g_run + padded
    lax.fori_loop(0, NB, pref_body, jnp.int32(0))

    def zero_idx(i, _):
        z = jnp.zeros((16,), jnp.int32)
        idxreg[pl.ds(i * 64, 16)] = z
        idxreg[pl.ds(i * 64 + 16, 16)] = z
        idxreg[pl.ds(i * 64 + 32, 16)] = z
        idxreg[pl.ds(i * 64 + 48, 16)] = z
        return 0
    lax.fori_loop(0, BSZ // 64, zero_idx, 0)

    def bucket_body(j, _):
        b = j * 32 + w
        start = pl.multiple_of(start_smem[j], 16)
        cnt = cnt_smem[j]
        tag = lax.shift_left(j + 1, 21)

        def zero_val(i, _):
            z = jnp.zeros((16,), jnp.float32)
            valreg[pl.ds(i * 64, 16)] = z
            valreg[pl.ds(i * 64 + 16, 16)] = z
            valreg[pl.ds(i * 64 + 32, 16)] = z
            valreg[pl.ds(i * 64 + 48, 16)] = z
            return 0
        lax.fori_loop(0, BSZ // 64, zero_val, 0)

        nch = lax.shift_right_logical(cnt + (CH3 - 1), 12)

        def chunk_a(c, _):
            pltpu.sync_copy(
                bpix_hbm.at[pl.ds(pl.multiple_of(start + c * CH3, 16), CH3)],
                pixbuf)

            def vec_body(i, _):
                pix = pixbuf[pl.ds(i * 16, 16)]
                pos = c * CH3 + i * 16 + iota
                pt = pos + tag
                valid = (pos < cnt) & (pix < PIX_SENT)
                pixc = pix & (BSZ - 1)

                def converge(carry):
                    wv = plsc.load_gather(idxreg, [pixc])
                    m = (pt > wv) & valid
                    plsc.store_scatter(idxreg, [pixc], pt, mask=m)
                    return jnp.max(plsc.all_reduce_population_count(m))
                lax.while_loop(lambda t: t > 0, converge, jnp.int32(1))
                return 0
            lax.fori_loop(0, CH3 // 16, vec_body, 0)
            return 0
        lax.fori_loop(0, nch, chunk_a, 0)

        def chunk_b(c, _):
            pltpu.sync_copy(
                bpix_hbm.at[pl.ds(pl.multiple_of(start + c * CH3, 16), CH3)],
                pixbuf)
            pltpu.sync_copy(
                bint_hbm.at[pl.ds(pl.multiple_of(start + c * CH3, 16), CH3)],
                intbuf)

            def vec_body(i, _):
                pix = pixbuf[pl.ds(i * 16, 16)]
                pos = c * CH3 + i * 16 + iota
                pt = pos + tag
                valid = (pos < cnt) & (pix < PIX_SENT)
                pixc = pix & (BSZ - 1)
                wv = plsc.load_gather(idxreg, [pixc])
                m = (wv == pt) & valid
                plsc.store_scatter(valreg, [pixc], intbuf[pl.ds(i * 16, 16)],
                                   mask=m)
                return 0
            lax.fori_loop(0, CH3 // 16, vec_body, 0)
            return 0
        lax.fori_loop(0, nch, chunk_b, 0)

        pltpu.sync_copy(valreg, out_hbm.at[pl.ds(b * BSZ, BSZ)])
        return 0
    lax.fori_loop(0, 16, bucket_body, 0)


def _sc_scatter(keys, inten_pad):
    mesh = plsc.VectorSubcoreMesh(core_axis_name="c", subcore_axis_name="s")
    params = pltpu.CompilerParams(needs_layout_passes=False,
                                  use_tc_tiling_on_sc=False)

    k1 = pl.kernel(
        _sc_count_body,
        out_type=[jax.ShapeDtypeStruct((NT * NB,), jnp.int32),
                  jax.ShapeDtypeStruct((NT * CNTW,), jnp.int32)],
        mesh=mesh,
        scratch_types=[pltpu.VMEM((CNTW,), jnp.int32),
                       pltpu.VMEM((NB,), jnp.int32),
                       pltpu.VMEM((CH,), jnp.int32)],
        compiler_params=params,
    )
    tb, cnt3 = k1(keys)

    k2 = pl.kernel(
        _sc_place_body,
        out_type=[jax.ShapeDtypeStruct((GROWS, 16), jnp.int32),
                  jax.ShapeDtypeStruct((GROWS, 16), jnp.float32)],
        mesh=mesh,
        scratch_types=[pltpu.VMEM((NT * NB,), jnp.int32),
                       pltpu.VMEM((640,), jnp.int32),
                       pltpu.VMEM((LROWS, 16), jnp.int32),
                       pltpu.VMEM((LROWS, 16), jnp.float32),
                       pltpu.VMEM((2560,), jnp.int32),
                       pltpu.VMEM((2482,), jnp.int32),
                       pltpu.VMEM((512,), jnp.int32),
                       pltpu.VMEM((CH,), jnp.int32),
                       pltpu.VMEM((CH,), jnp.float32),
                       pltpu.VMEM((128,), jnp.int32),
                       pltpu.SemaphoreType.DMA],
        compiler_params=params,
    )
    bpix, bint = k2(keys, inten_pad, tb)
    bpix = bpix.reshape(-1)
    bint = bint.reshape(-1)

    k3 = pl.kernel(
        _sc_emit_body,
        out_type=jax.ShapeDtypeStruct((SENT,), jnp.float32),
        mesh=mesh,
        scratch_types=[pltpu.VMEM((NT * NB,), jnp.int32),
                       pltpu.VMEM((BSZ,), jnp.int32),
                       pltpu.VMEM((BSZ,), jnp.float32),
                       pltpu.VMEM((CH3,), jnp.int32),
                       pltpu.VMEM((CH3,), jnp.float32),
                       pltpu.SMEM((16,), jnp.int32),
                       pltpu.SMEM((16,), jnp.int32)],
        compiler_params=params,
    )
    return k3(bpix, bint, tb)


def kernel(points, cam_params):
    keys, inten_pad = _compute_keys(points, cam_params)
    buf = _sc_scatter(keys, inten_pad)
    front = buf[0:S * S].reshape(S, S)
    back = buf[S * S:2 * S * S].reshape(S, S)
    left = buf[2 * S * S:3 * S * S].reshape(S, S)
    right = buf[3 * S * S:4 * S * S].reshape(S, S)
    return front, back, left, right


# speedup vs baseline: 1.0126x; 1.0126x over previous
import functools

import jax
import jax.numpy as jnp
from jax import lax
from jax.experimental import pallas as pl
from jax.experimental.pallas import tpu as pltpu
from jax.experimental.pallas import tpu_sc as plsc

S = 2048
SENT = 4 * S * S
_A = 6378137.0
_E2 = 6.69437999014e-3

_LANES = 1024
_RBLK = 8

NT = 32
NB = 512
BSZ = SENT // NB
VSHIFT = 15
NPAD = 1007616
PPT = NPAD // NT
CH = 3936
NCH = PPT // CH
CH3 = 4096
CNTW = 8320
LROWS = 2482
MG = 1273888
GROWS = MG // 16
PIX_SENT = 32768


def _transform_body(scal_ref, pts_ref, key_ref, int_ref, *, n_valid, rows_per_blk):
    lat = pts_ref[0]
    lon = pts_ref[1]
    alt = pts_ref[2]

    x0 = scal_ref[0]
    y0 = scal_ref[1]
    z0 = scal_ref[2]
    neg_so = scal_ref[3]
    co = scal_ref[4]
    neg_sl_co = scal_ref[5]
    sl_so = scal_ref[6]
    cl = scal_ref[7]
    cl_co = scal_ref[8]
    cl_so = scal_ref[9]
    sl = scal_ref[10]
    r00 = scal_ref[11]
    r01 = scal_ref[12]
    r02 = scal_ref[13]
    r10 = scal_ref[14]
    r11 = scal_ref[15]
    r12 = scal_ref[16]
    r20 = scal_ref[17]
    r21 = scal_ref[18]
    r22 = scal_ref[19]

    latr = jnp.deg2rad(lat)
    lonr = jnp.deg2rad(lon)
    s = jnp.sin(latr)
    c = jnp.cos(latr)
    Nv = _A * lax.rsqrt(1.0 - _E2 * s * s)
    X = (Nv + alt) * c * jnp.cos(lonr)
    Y = (Nv + alt) * c * jnp.sin(lonr)
    Z = (Nv * (1.0 - _E2) + alt) * s

    dx = X - x0
    dy = Y - y0
    dz = Z - z0
    e = neg_so * dx + co * dy
    n = neg_sl_co * dx - sl_so * dy + cl * dz
    u = cl_co * dx + cl_so * dy + sl * dz

    x = r00 * e + r01 * n + r02 * u
    y = r10 * e + r11 * n + r12 * u
    z = r20 * e + r21 * n + r22 * u

    ax = jnp.abs(x)
    ay = jnp.abs(y)
    az = jnp.abs(z)
    m_front = (z > 0) & (z > ax) & (z > ay)
    m_back = (z < 0) & (-z > ax) & (-z > ay)
    m_right = (x > 0) & (x > az) & (x > ay)
    m_left = (x < 0) & (-x > az) & (-x > ay)

    f = S / 2.0

    def cam2key(px_x, px_y, px_z):
        z_safe = jnp.where(jnp.abs(px_z) > 1e-9, px_z, 1.0)
        pu = f * px_x / z_safe + f
        pv = f * px_y / z_safe + f
        iu = jnp.clip(jnp.floor(pu), 0, S - 1).astype(jnp.int32)
        iv = jnp.clip(jnp.floor(pv), 0, S - 1).astype(jnp.int32)
        return iu * S + iv

    kf = cam2key(x, y, z)
    kb = cam2key(x, -y, z)
    kr = cam2key(-z, y, x)
    kl = cam2key(z, y, -x)

    key = jnp.where(
        m_front, kf,
        jnp.where(m_back, S * S + kb,
                  jnp.where(m_left, 2 * S * S + kl,
                            jnp.where(m_right, 3 * S * S + kr, SENT))))

    pid = pl.program_id(0)
    row = lax.broadcasted_iota(jnp.int32, (rows_per_blk, _LANES), 0)
    col = lax.broadcasted_iota(jnp.int32, (rows_per_blk, _LANES), 1)
    gidx = (pid * rows_per_blk + row) * _LANES + col
    key_ref[...] = jnp.where(gidx < n_valid, key, SENT)
    int_ref[...] = pts_ref[3]


def _compute_keys(points, cam_params):
    n = points.shape[0]
    rows = NPAD // _LANES
    grid = rows // _RBLK

    pts_t = jnp.transpose(points)
    pts_t = jnp.pad(pts_t, ((0, 0), (0, NPAD - n)))
    pts_t = pts_t.reshape(4, rows, _LANES)

    lat0, lon0, alt0 = cam_params[0], cam_params[1], cam_params[2]
    latr0 = jnp.deg2rad(lat0)
    lonr0 = jnp.deg2rad(lon0)
    sl = jnp.sin(latr0)
    cl = jnp.cos(latr0)
    so = jnp.sin(lonr0)
    co = jnp.cos(lonr0)
    Nv0 = _A / jnp.sqrt(1.0 - _E2 * sl * sl)
    x0 = (Nv0 + alt0) * cl * jnp.cos(lonr0)
    y0 = (Nv0 + alt0) * cl * jnp.sin(lonr0)
    z0 = (Nv0 * (1.0 - _E2) + alt0) * sl

    qs = -cam_params[3]
    qx = cam_params[4]
    qy = cam_params[5]
    qz = cam_params[6]
    nrm = jnp.sqrt(qs * qs + qx * qx + qy * qy + qz * qz) + 1e-12
    qs, qx, qy, qz = qs / nrm, qx / nrm, qy / nrm, qz / nrm
    r00 = 1 - 2 * (qy * qy + qz * qz)
    r01 = 2 * (qx * qy - qz * qs)
    r02 = 2 * (qx * qz + qy * qs)
    r10 = 2 * (qx * qy + qz * qs)
    r11 = 1 - 2 * (qx * qx + qz * qz)
    r12 = 2 * (qy * qz - qx * qs)
    r20 = 2 * (qx * qz - qy * qs)
    r21 = 2 * (qy * qz + qx * qs)
    r22 = 1 - 2 * (qx * qx + qy * qy)

    scal = jnp.stack([
        x0, y0, z0, -so, co, -sl * co, sl * so, cl, cl * co, cl * so, sl,
        r00, r01, r02, r10, r11, r12, r20, r21, r22,
    ]).astype(jnp.float32)

    keys, inten = pl.pallas_call(
        functools.partial(_transform_body, n_valid=n, rows_per_blk=_RBLK),
        grid=(grid,),
        in_specs=[
            pl.BlockSpec(memory_space=pltpu.SMEM),
            pl.BlockSpec((4, _RBLK, _LANES), lambda i: (0, i, 0)),
        ],
        out_specs=[pl.BlockSpec((_RBLK, _LANES), lambda i: (i, 0)),
                   pl.BlockSpec((_RBLK, _LANES), lambda i: (i, 0))],
        out_shape=[jax.ShapeDtypeStruct((rows, _LANES), jnp.int32),
                   jax.ShapeDtypeStruct((rows, _LANES), jnp.float32)],
    )(scal, pts_t)
    return keys.reshape(-1), inten.reshape(-1)


def _wid():
    return lax.axis_index("s") * 2 + lax.axis_index("c")


_IOTA = lambda: lax.broadcasted_iota(jnp.int32, (16,), 0)


def _sc_count_body(keys_hbm, tb_hbm, cnt3_hbm, cnt3_v, tbrow_v, keybuf):
    w = _wid()
    iota = _IOTA()

    def zero_body(i, _):
        cnt3_v[pl.ds(i * 16, 16)] = jnp.zeros((16,), jnp.int32)
        return 0
    lax.fori_loop(0, CNTW // 16, zero_body, 0)

    def chunk_body(c, _):
        pltpu.sync_copy(keys_hbm.at[pl.ds(w * PPT + c * CH, CH)], keybuf)

        def vec_body(i, _):
            k = keybuf[pl.ds(i * 16, 16)]
            b = lax.shift_right_logical(k, VSHIFT)
            cidx = iota * 513 + b
            cv = plsc.load_gather(cnt3_v, [cidx])
            plsc.store_scatter(cnt3_v, [cidx], cv + 1)
            return 0
        lax.fori_loop(0, CH // 16, vec_body, 0)
        return 0
    lax.fori_loop(0, NCH, chunk_body, 0)

    def tb_body(bc, _):
        acc = jnp.zeros((16,), jnp.int32)
        for l in range(16):
            acc = acc + cnt3_v[pl.ds(l * 513 + bc * 16, 16)]
        tbrow_v[pl.ds(bc * 16, 16)] = acc
        return 0
    lax.fori_loop(0, NB // 16, tb_body, 0)

    pltpu.sync_copy(tbrow_v, tb_hbm.at[pl.ds(w * NB, NB)])
    pltpu.sync_copy(cnt3_v, cnt3_hbm.at[pl.ds(w * CNTW, CNTW)])


def _sc_place_body(keys_hbm, inten_hbm, tb_hbm,
                   bpix_hbm, bint_hbm,
                   tb_v, cnt1_v, pix_v, intl_v, rowb_v, rowidx_v, shift_v,
                   keybuf, intbuf, s16_v, sem):
    w = _wid()
    iota = _IOTA()
    pltpu.sync_copy(tb_hbm, tb_v)

    def zero_rowb(i, _):
        rowb_v[pl.ds(i * 16, 16)] = jnp.zeros((16,), jnp.int32)
        return 0
    lax.fori_loop(0, 2560 // 16, zero_rowb, 0)

    lane0 = _IOTA() == 0

    def base_body(b, carry):
        g_run, lb_run = carry
        tot = jnp.max(plsc.load_gather(tb_v, [jnp.full((16,), w * NB + b,
                                                       jnp.int32)]))
        plsc.store_scatter(cnt1_v, [jnp.full((16,), b, jnp.int32)],
                           jnp.full((16,), lb_run, jnp.int32), mask=lane0)
        c1 = plsc.load_gather(tb_v, [iota * NB + b])
        c2 = plsc.load_gather(tb_v, [(iota + 16) * NB + b])
        p1 = (c1 + 15) & -16
        p2 = (c2 + 15) & -16
        zero = jnp.zeros((16,), jnp.int32)
        cross = (jnp.sum(jnp.where(iota < w, p1, zero)) +
                 jnp.sum(jnp.where(iota < w - 16, p2, zero)))
        rstart = lax.shift_right_logical(lb_run, 4)
        plsc.store_scatter(rowb_v, [jnp.full((16,), rstart, jnp.int32)],
                           jnp.full((16,), b, jnp.int32), mask=lane0)
        shift = lax.shift_right_logical((g_run + cross) - lb_run, 4)
        plsc.store_scatter(shift_v, [jnp.full((16,), b, jnp.int32)],
                           jnp.full((16,), shift, jnp.int32), mask=lane0)
        g_next = g_run + (jnp.sum(p1) + jnp.sum(p2))
        lb_next = lb_run + ((tot + 15) & -16)
        return (g_next, lb_next)
    _, lb_total = lax.fori_loop(0, NB, base_body,
                                (jnp.int32(0), jnp.int32(0)))

    def fill_body(r, _):
        plsc.store_scatter(pix_v, [jnp.full((16,), r, jnp.int32), iota],
                           jnp.full((16,), PIX_SENT, jnp.int32))
        return 0
    lax.fori_loop(0, LROWS, fill_body, 0)

    def chunk_body(c, _):
        off = w * PPT + c * CH
        pltpu.sync_copy(keys_hbm.at[pl.ds(off, CH)], keybuf)
        pltpu.sync_copy(inten_hbm.at[pl.ds(off, CH)], intbuf)

        def vec_body(i, _):
            k = keybuf[pl.ds(i * 16, 16)]
            b = lax.shift_right_logical(k, VSHIFT)
            valid = b < NB
            ck = b * 16 + iota
            sk, perm = plsc.sort_key_val(ck, iota)
            bs = lax.shift_right_logical(sk, 4)
            prev = bs.at[jnp.maximum(iota - 1, 0)].get(
                mode="promise_in_bounds")
            nxt = bs.at[jnp.minimum(iota + 1, 15)].get(
                mode="promise_in_bounds")
            zero = jnp.zeros((16,), jnp.int32)
            neq0 = (iota == 0) | (bs != prev)
            startpos = plsc.cummax(jnp.where(neq0, iota, zero))
            crank = iota - startpos
            cv = plsc.load_gather(cnt1_v, [bs])
            slot_sorted = cv + crank
            islast = (iota == 15) | (bs != nxt)
            plsc.store_scatter(cnt1_v, [bs], slot_sorted + 1, mask=islast)
            pix_s = (k & (BSZ - 1)).at[perm].get(mode="promise_in_bounds")
            int_s = intbuf[pl.ds(i * 16, 16)].at[perm].get(
                mode="promise_in_bounds")
            valid_s = sk < NB * 16
            r = jnp.where(valid_s, lax.shift_right_logical(slot_sorted, 4), 0)
            col = jnp.where(valid_s, slot_sorted & 15, 0)
            plsc.store_scatter(pix_v, [r, col], pix_s, mask=valid_s)
            plsc.store_scatter(intl_v, [r, col], int_s, mask=valid_s)
            return 0
        lax.fori_loop(0, CH // 16, vec_body, 0)
        return 0
    lax.fori_loop(0, NCH, chunk_body, 0)

    used_rows = lax.shift_right_logical(lb_total, 4)

    def row_body(i, carry):
        rb = rowb_v[pl.ds(i * 16, 16)]
        cm = jnp.maximum(plsc.cummax(rb), jnp.full((16,), carry, jnp.int32))
        sh = plsc.load_gather(shift_v, [cm])
        lrow = i * 16 + iota
        gr = jnp.where(lrow < used_rows, lrow + sh, GROWS - 1)
        rowidx_v[pl.ds(i * 16, 16)] = gr
        return jnp.max(cm)
    lax.fori_loop(0, (LROWS + 15) // 16, row_body, jnp.int32(0))

    c1 = pltpu.async_copy(pix_v, bpix_hbm.at[rowidx_v], sem)
    c2 = pltpu.async_copy(intl_v, bint_hbm.at[rowidx_v], sem)
    c1.wait()
    c2.wait()


def _sc_emit_body(bpix_hbm, bint_hbm, tb_hbm, out_hbm,
                  tb_v, idxreg, valreg, pixbuf, intbuf,
                  start_smem, cnt_smem):
    w = _wid()
    iota = _IOTA()
    pltpu.sync_copy(tb_hbm, tb_v)

    def pref_body(b, g_run):
        c1 = plsc.load_gather(tb_v, [iota * NB + b])
        c2 = plsc.load_gather(tb_v, [(iota + 16) * NB + b])
        p1 = (c1 + 15) & -16
        p2 = (c2 + 15) & -16
        padded = jnp.sum(p1) + jnp.sum(p2)
        j = lax.shift_right_logical(b, 5)

        @pl.when((b & 31) == w)
        def _():
            start_smem[j] = g_run
            cnt_smem[j] = padded
        return g_run + padded
    lax.fori_loop(0, NB, pref_body, jnp.int32(0))

    def zero_idx(i, _):
        z = jnp.zeros((16,), jnp.int32)
        idxreg[pl.ds(i * 64, 16)] = z
        idxreg[pl.ds(i * 64 + 16, 16)] = z
        idxreg[pl.ds(i * 64 + 32, 16)] = z
        idxreg[pl.ds(i * 64 + 48, 16)] = z
        return 0
    lax.fori_loop(0, BSZ // 64, zero_idx, 0)

    def bucket_body(j, _):
        b = j * 32 + w
        start = pl.multiple_of(start_smem[j], 16)
        cnt = cnt_smem[j]
        tag = lax.shift_left(j + 1, 21)

        def zero_val(i, _):
            z = jnp.zeros((16,), jnp.float32)
            valreg[pl.ds(i * 64, 16)] = z
            valreg[pl.ds(i * 64 + 16, 16)] = z
            valreg[pl.ds(i * 64 + 32, 16)] = z
            valreg[pl.ds(i * 64 + 48, 16)] = z
            return 0
        lax.fori_loop(0, BSZ // 64, zero_val, 0)

        nch = lax.shift_right_logical(cnt + (CH3 - 1), 12)

        def chunk_a(c, _):
            pltpu.sync_copy(
                bpix_hbm.at[pl.ds(pl.multiple_of(start + c * CH3, 16), CH3)],
                pixbuf)

            def vec_body(i, _):
                pix = pixbuf[pl.ds(i * 16, 16)]
                pos = c * CH3 + i * 16 + iota
                pt = pos + tag
                valid = (pos < cnt) & (pix < PIX_SENT)
                pixc = pix & (BSZ - 1)

                def converge(carry):
                    wv = plsc.load_gather(idxreg, [pixc])
                    m = (pt > wv) & valid
                    plsc.store_scatter(idxreg, [pixc], pt, mask=m)
                    return jnp.max(plsc.all_reduce_population_count(m))
                lax.while_loop(lambda t: t > 0, converge, jnp.int32(1))
                return 0
            lax.fori_loop(0, CH3 // 16, vec_body, 0)
            return 0
        lax.fori_loop(0, nch, chunk_a, 0)

        def chunk_b(c, _):
            pltpu.sync_copy(
                bpix_hbm.at[pl.ds(pl.multiple_of(start + c * CH3, 16), CH3)],
                pixbuf)
            pltpu.sync_copy(
                bint_hbm.at[pl.ds(pl.multiple_of(start + c * CH3, 16), CH3)],
                intbuf)

            def vec_body(i, _):
                pix = pixbuf[pl.ds(i * 16, 16)]
                pos = c * CH3 + i * 16 + iota
                pt = pos + tag
                valid = (pos < cnt) & (pix < PIX_SENT)
                pixc = pix & (BSZ - 1)
                wv = plsc.load_gather(idxreg, [pixc])
                m = (wv == pt) & valid
                plsc.store_scatter(valreg, [pixc], intbuf[pl.ds(i * 16, 16)],
                                   mask=m)
                return 0
            lax.fori_loop(0, CH3 // 16, vec_body, 0)
            return 0
        lax.fori_loop(0, nch, chunk_b, 0)

        pltpu.sync_copy(valreg, out_hbm.at[pl.ds(b * BSZ, BSZ)])
        return 0
    lax.fori_loop(0, 16, bucket_body, 0)


def _sc_scatter(keys, inten_pad):
    mesh = plsc.VectorSubcoreMesh(core_axis_name="c", subcore_axis_name="s")
    params = pltpu.CompilerParams(needs_layout_passes=False,
                                  use_tc_tiling_on_sc=False)

    k1 = pl.kernel(
        _sc_count_body,
        out_type=[jax.ShapeDtypeStruct((NT * NB,), jnp.int32),
                  jax.ShapeDtypeStruct((NT * CNTW,), jnp.int32)],
        mesh=mesh,
        scratch_types=[pltpu.VMEM((CNTW,), jnp.int32),
                       pltpu.VMEM((NB,), jnp.int32),
                       pltpu.VMEM((CH,), jnp.int32)],
        compiler_params=params,
    )
    tb, cnt3 = k1(keys)

    k2 = pl.kernel(
        _sc_place_body,
        out_type=[jax.ShapeDtypeStruct((GROWS, 16), jnp.int32),
                  jax.ShapeDtypeStruct((GROWS, 16), jnp.float32)],
        mesh=mesh,
        scratch_types=[pltpu.VMEM((NT * NB,), jnp.int32),
                       pltpu.VMEM((640,), jnp.int32),
                       pltpu.VMEM((LROWS, 16), jnp.int32),
                       pltpu.VMEM((LROWS, 16), jnp.float32),
                       pltpu.VMEM((2560,), jnp.int32),
                       pltpu.VMEM((2482,), jnp.int32),
                       pltpu.VMEM((512,), jnp.int32),
                       pltpu.VMEM((CH,), jnp.int32),
                       pltpu.VMEM((CH,), jnp.float32),
                       pltpu.VMEM((128,), jnp.int32),
                       pltpu.SemaphoreType.DMA],
        compiler_params=params,
    )
    bpix, bint = k2(keys, inten_pad, tb)
    bpix = bpix.reshape(-1)
    bint = bint.reshape(-1)

    k3 = pl.kernel(
        _sc_emit_body,
        out_type=jax.ShapeDtypeStruct((SENT,), jnp.float32),
        mesh=mesh,
        scratch_types=[pltpu.VMEM((NT * NB,), jnp.int32),
                       pltpu.VMEM((BSZ,), jnp.int32),
                       pltpu.VMEM((BSZ,), jnp.float32),
                       pltpu.VMEM((CH3,), jnp.int32),
                       pltpu.VMEM((CH3,), jnp.float32),
                       pltpu.SMEM((16,), jnp.int32),
                       pltpu.SMEM((16,), jnp.int32)],
        compiler_params=params,
    )
    return k3(bpix, bint, tb)


def kernel(points, cam_params):
    keys, inten_pad = _compute_keys(points, cam_params)
    buf = _sc_scatter(keys, inten_pad)
    front = buf[0:S * S].reshape(S, S)
    back = buf[S * S:2 * S * S].reshape(S, S)
    left = buf[2 * S * S:3 * S * S].reshape(S, S)
    right = buf[3 * S * S:4 * S * S].reshape(S, S)
    return front, back, left, right


# speedup vs baseline: 1.0182x; 1.0055x over previous
import functools

import jax
import jax.numpy as jnp
from jax import lax
from jax.experimental import pallas as pl
from jax.experimental.pallas import tpu as pltpu
from jax.experimental.pallas import tpu_sc as plsc

S = 2048
SENT = 4 * S * S
_A = 6378137.0
_E2 = 6.69437999014e-3

_LANES = 1024
_RBLK = 8

NT = 32
NB = 512
BSZ = SENT // NB
VSHIFT = 15
NPAD = 1007616
PPT = NPAD // NT
CH = 3936
NCH = PPT // CH
CH3 = 4096
CNTW = 8320
LROWS = 2482
MG = 1273888
GROWS = MG // 16
PIX_SENT = 32768


def _transform_body(scal_ref, pts_ref, key_ref, int_ref, *, n_valid, rows_per_blk):
    lat = pts_ref[0]
    lon = pts_ref[1]
    alt = pts_ref[2]

    x0 = scal_ref[0]
    y0 = scal_ref[1]
    z0 = scal_ref[2]
    neg_so = scal_ref[3]
    co = scal_ref[4]
    neg_sl_co = scal_ref[5]
    sl_so = scal_ref[6]
    cl = scal_ref[7]
    cl_co = scal_ref[8]
    cl_so = scal_ref[9]
    sl = scal_ref[10]
    r00 = scal_ref[11]
    r01 = scal_ref[12]
    r02 = scal_ref[13]
    r10 = scal_ref[14]
    r11 = scal_ref[15]
    r12 = scal_ref[16]
    r20 = scal_ref[17]
    r21 = scal_ref[18]
    r22 = scal_ref[19]

    latr = jnp.deg2rad(lat)
    lonr = jnp.deg2rad(lon)
    s = jnp.sin(latr)
    c = jnp.cos(latr)
    Nv = _A * lax.rsqrt(1.0 - _E2 * s * s)
    X = (Nv + alt) * c * jnp.cos(lonr)
    Y = (Nv + alt) * c * jnp.sin(lonr)
    Z = (Nv * (1.0 - _E2) + alt) * s

    dx = X - x0
    dy = Y - y0
    dz = Z - z0
    e = neg_so * dx + co * dy
    n = neg_sl_co * dx - sl_so * dy + cl * dz
    u = cl_co * dx + cl_so * dy + sl * dz

    x = r00 * e + r01 * n + r02 * u
    y = r10 * e + r11 * n + r12 * u
    z = r20 * e + r21 * n + r22 * u

    ax = jnp.abs(x)
    ay = jnp.abs(y)
    az = jnp.abs(z)
    m_front = (z > 0) & (z > ax) & (z > ay)
    m_back = (z < 0) & (-z > ax) & (-z > ay)
    m_right = (x > 0) & (x > az) & (x > ay)
    m_left = (x < 0) & (-x > az) & (-x > ay)

    f = S / 2.0

    def cam2key(px_x, px_y, px_z):
        z_safe = jnp.where(jnp.abs(px_z) > 1e-9, px_z, 1.0)
        pu = f * px_x / z_safe + f
        pv = f * px_y / z_safe + f
        iu = jnp.clip(jnp.floor(pu), 0, S - 1).astype(jnp.int32)
        iv = jnp.clip(jnp.floor(pv), 0, S - 1).astype(jnp.int32)
        return iu * S + iv

    kf = cam2key(x, y, z)
    kb = cam2key(x, -y, z)
    kr = cam2key(-z, y, x)
    kl = cam2key(z, y, -x)

    key = jnp.where(
        m_front, kf,
        jnp.where(m_back, S * S + kb,
                  jnp.where(m_left, 2 * S * S + kl,
                            jnp.where(m_right, 3 * S * S + kr, SENT))))

    pid = pl.program_id(0)
    row = lax.broadcasted_iota(jnp.int32, (rows_per_blk, _LANES), 0)
    col = lax.broadcasted_iota(jnp.int32, (rows_per_blk, _LANES), 1)
    gidx = (pid * rows_per_blk + row) * _LANES + col
    key_ref[...] = jnp.where(gidx < n_valid, key, SENT)
    int_ref[...] = pts_ref[3]


def _compute_keys(points, cam_params):
    n = points.shape[0]
    rows = NPAD // _LANES
    grid = rows // _RBLK

    pts_t = jnp.transpose(points)
    pts_t = jnp.pad(pts_t, ((0, 0), (0, NPAD - n)))
    pts_t = pts_t.reshape(4, rows, _LANES)

    lat0, lon0, alt0 = cam_params[0], cam_params[1], cam_params[2]
    latr0 = jnp.deg2rad(lat0)
    lonr0 = jnp.deg2rad(lon0)
    sl = jnp.sin(latr0)
    cl = jnp.cos(latr0)
    so = jnp.sin(lonr0)
    co = jnp.cos(lonr0)
    Nv0 = _A / jnp.sqrt(1.0 - _E2 * sl * sl)
    x0 = (Nv0 + alt0) * cl * jnp.cos(lonr0)
    y0 = (Nv0 + alt0) * cl * jnp.sin(lonr0)
    z0 = (Nv0 * (1.0 - _E2) + alt0) * sl

    qs = -cam_params[3]
    qx = cam_params[4]
    qy = cam_params[5]
    qz = cam_params[6]
    nrm = jnp.sqrt(qs * qs + qx * qx + qy * qy + qz * qz) + 1e-12
    qs, qx, qy, qz = qs / nrm, qx / nrm, qy / nrm, qz / nrm
    r00 = 1 - 2 * (qy * qy + qz * qz)
    r01 = 2 * (qx * qy - qz * qs)
    r02 = 2 * (qx * qz + qy * qs)
    r10 = 2 * (qx * qy + qz * qs)
    r11 = 1 - 2 * (qx * qx + qz * qz)
    r12 = 2 * (qy * qz - qx * qs)
    r20 = 2 * (qx * qz - qy * qs)
    r21 = 2 * (qy * qz + qx * qs)
    r22 = 1 - 2 * (qx * qx + qy * qy)

    scal = jnp.stack([
        x0, y0, z0, -so, co, -sl * co, sl * so, cl, cl * co, cl * so, sl,
        r00, r01, r02, r10, r11, r12, r20, r21, r22,
    ]).astype(jnp.float32)

    keys, inten = pl.pallas_call(
        functools.partial(_transform_body, n_valid=n, rows_per_blk=_RBLK),
        grid=(grid,),
        in_specs=[
            pl.BlockSpec(memory_space=pltpu.SMEM),
            pl.BlockSpec((4, _RBLK, _LANES), lambda i: (0, i, 0)),
        ],
        out_specs=[pl.BlockSpec((_RBLK, _LANES), lambda i: (i, 0)),
                   pl.BlockSpec((_RBLK, _LANES), lambda i: (i, 0))],
        out_shape=[jax.ShapeDtypeStruct((rows, _LANES), jnp.int32),
                   jax.ShapeDtypeStruct((rows, _LANES), jnp.float32)],
    )(scal, pts_t)
    return keys.reshape(-1), inten.reshape(-1)


def _wid():
    return lax.axis_index("s") * 2 + lax.axis_index("c")


_IOTA = lambda: lax.broadcasted_iota(jnp.int32, (16,), 0)


def _sc_count_body(keys_hbm, tb_hbm, cnt3_hbm, cnt3_v, tbrow_v, keybuf):
    w = _wid()
    iota = _IOTA()

    def zero_body(i, _):
        cnt3_v[pl.ds(i * 16, 16)] = jnp.zeros((16,), jnp.int32)
        return 0
    lax.fori_loop(0, CNTW // 16, zero_body, 0)

    def chunk_body(c, _):
        pltpu.sync_copy(keys_hbm.at[pl.ds(w * PPT + c * CH, CH)], keybuf)

        def vec_body(i, _):
            k = keybuf[pl.ds(i * 16, 16)]
            b = lax.shift_right_logical(k, VSHIFT)
            cidx = iota * 513 + b
            cv = plsc.load_gather(cnt3_v, [cidx])
            plsc.store_scatter(cnt3_v, [cidx], cv + 1)
            return 0
        lax.fori_loop(0, CH // 16, vec_body, 0)
        return 0
    lax.fori_loop(0, NCH, chunk_body, 0)

    def tb_body(bc, _):
        acc = jnp.zeros((16,), jnp.int32)
        for l in range(16):
            acc = acc + cnt3_v[pl.ds(l * 513 + bc * 16, 16)]
        tbrow_v[pl.ds(bc * 16, 16)] = acc
        return 0
    lax.fori_loop(0, NB // 16, tb_body, 0)

    pltpu.sync_copy(tbrow_v, tb_hbm.at[pl.ds(w * NB, NB)])
    pltpu.sync_copy(cnt3_v, cnt3_hbm.at[pl.ds(w * CNTW, CNTW)])


def _sc_place_body(keys_hbm, inten_hbm, tb_hbm,
                   bpix_hbm, bint_hbm,
                   tb_v, cnt1_v, pix_v, intl_v, rowb_v, rowidx_v, shift_v,
                   keybuf, intbuf, s16_v, sem):
    w = _wid()
    iota = _IOTA()
    pltpu.sync_copy(tb_hbm, tb_v)

    def zero_rowb(i, _):
        rowb_v[pl.ds(i * 16, 16)] = jnp.zeros((16,), jnp.int32)
        return 0
    lax.fori_loop(0, 2560 // 16, zero_rowb, 0)

    lane0 = _IOTA() == 0

    def base_body(b, carry):
        g_run, lb_run = carry
        tot = jnp.max(plsc.load_gather(tb_v, [jnp.full((16,), w * NB + b,
                                                       jnp.int32)]))
        plsc.store_scatter(cnt1_v, [jnp.full((16,), b, jnp.int32)],
                           jnp.full((16,), lb_run, jnp.int32), mask=lane0)
        c1 = plsc.load_gather(tb_v, [iota * NB + b])
        c2 = plsc.load_gather(tb_v, [(iota + 16) * NB + b])
        p1 = (c1 + 15) & -16
        p2 = (c2 + 15) & -16
        zero = jnp.zeros((16,), jnp.int32)
        cross = (jnp.sum(jnp.where(iota < w, p1, zero)) +
                 jnp.sum(jnp.where(iota < w - 16, p2, zero)))
        rstart = lax.shift_right_logical(lb_run, 4)
        plsc.store_scatter(rowb_v, [jnp.full((16,), rstart, jnp.int32)],
                           jnp.full((16,), b, jnp.int32), mask=lane0)
        shift = lax.shift_right_logical((g_run + cross) - lb_run, 4)
        plsc.store_scatter(shift_v, [jnp.full((16,), b, jnp.int32)],
                           jnp.full((16,), shift, jnp.int32), mask=lane0)
        g_next = g_run + (jnp.sum(p1) + jnp.sum(p2))
        lb_next = lb_run + ((tot + 15) & -16)
        return (g_next, lb_next)
    _, lb_total = lax.fori_loop(0, NB, base_body,
                                (jnp.int32(0), jnp.int32(0)))

    def fill_body(r, _):
        plsc.store_scatter(pix_v, [jnp.full((16,), r, jnp.int32), iota],
                           jnp.full((16,), PIX_SENT, jnp.int32))
        return 0
    lax.fori_loop(0, LROWS, fill_body, 0)

    def chunk_body(c, _):
        off = w * PPT + c * CH
        pltpu.sync_copy(keys_hbm.at[pl.ds(off, CH)], keybuf)
        pltpu.sync_copy(inten_hbm.at[pl.ds(off, CH)], intbuf)

        def vec_body(i, _):
            k = keybuf[pl.ds(i * 16, 16)]
            b = lax.shift_right_logical(k, VSHIFT)
            valid = b < NB
            ck = b * 16 + iota
            sk, perm = plsc.sort_key_val(ck, iota)
            bs = lax.shift_right_logical(sk, 4)
            prev = bs.at[jnp.maximum(iota - 1, 0)].get(
                mode="promise_in_bounds")
            nxt = bs.at[jnp.minimum(iota + 1, 15)].get(
                mode="promise_in_bounds")
            zero = jnp.zeros((16,), jnp.int32)
            neq0 = (iota == 0) | (bs != prev)
            startpos = plsc.cummax(jnp.where(neq0, iota, zero))
            crank = iota - startpos
            cv = plsc.load_gather(cnt1_v, [bs])
            slot_sorted = cv + crank
            islast = (iota == 15) | (bs != nxt)
            plsc.store_scatter(cnt1_v, [bs], slot_sorted + 1, mask=islast)
            pix_s = (k & (BSZ - 1)).at[perm].get(mode="promise_in_bounds")
            int_s = intbuf[pl.ds(i * 16, 16)].at[perm].get(
                mode="promise_in_bounds")
            valid_s = sk < NB * 16
            r = jnp.where(valid_s, lax.shift_right_logical(slot_sorted, 4), 0)
            col = jnp.where(valid_s, slot_sorted & 15, 0)
            plsc.store_scatter(pix_v, [r, col], pix_s, mask=valid_s)
            plsc.store_scatter(intl_v, [r, col], int_s, mask=valid_s)
            return 0
        lax.fori_loop(0, CH // 16, vec_body, 0)
        return 0
    lax.fori_loop(0, NCH, chunk_body, 0)

    used_rows = lax.shift_right_logical(lb_total, 4)

    def row_body(i, carry):
        rb = rowb_v[pl.ds(i * 16, 16)]
        cm = jnp.maximum(plsc.cummax(rb), jnp.full((16,), carry, jnp.int32))
        sh = plsc.load_gather(shift_v, [cm])
        lrow = i * 16 + iota
        gr = jnp.where(lrow < used_rows, lrow + sh, GROWS - 1)
        rowidx_v[pl.ds(i * 16, 16)] = gr
        return jnp.max(cm)
    lax.fori_loop(0, (LROWS + 15) // 16, row_body, jnp.int32(0))

    c1 = pltpu.async_copy(pix_v, bpix_hbm.at[rowidx_v], sem)
    c2 = pltpu.async_copy(intl_v, bint_hbm.at[rowidx_v], sem)
    c1.wait()
    c2.wait()


def _sc_emit_body(bpix_hbm, bint_hbm, tb_hbm, out_hbm,
                  tb_v, idxreg, valreg, pixbuf, intbuf,
                  start_smem, cnt_smem):
    w = _wid()
    iota = _IOTA()
    pltpu.sync_copy(tb_hbm, tb_v)

    def pref_body(b, g_run):
        c1 = plsc.load_gather(tb_v, [iota * NB + b])
        c2 = plsc.load_gather(tb_v, [(iota + 16) * NB + b])
        p1 = (c1 + 15) & -16
        p2 = (c2 + 15) & -16
        padded = jnp.sum(p1) + jnp.sum(p2)
        j = lax.shift_right_logical(b, 5)

        @pl.when((b & 31) == w)
        def _():
            start_smem[j] = g_run
            cnt_smem[j] = padded
        return g_run + padded
    lax.fori_loop(0, NB, pref_body, jnp.int32(0))

    def zero_idx(i, _):
        z = jnp.zeros((16,), jnp.int32)
        idxreg[pl.ds(i * 64, 16)] = z
        idxreg[pl.ds(i * 64 + 16, 16)] = z
        idxreg[pl.ds(i * 64 + 32, 16)] = z
        idxreg[pl.ds(i * 64 + 48, 16)] = z
        return 0
    lax.fori_loop(0, BSZ // 64, zero_idx, 0)

    def bucket_body(j, _):
        b = j * 32 + w
        start = pl.multiple_of(start_smem[j], 16)
        cnt = cnt_smem[j]
        tag = lax.shift_left(j + 1, 21)

        def zero_val(i, _):
            z = jnp.zeros((16,), jnp.float32)
            valreg[pl.ds(i * 64, 16)] = z
            valreg[pl.ds(i * 64 + 16, 16)] = z
            valreg[pl.ds(i * 64 + 32, 16)] = z
            valreg[pl.ds(i * 64 + 48, 16)] = z
            return 0
        lax.fori_loop(0, BSZ // 64, zero_val, 0)

        nch = lax.shift_right_logical(cnt + (CH3 - 1), 12)

        def chunk_a(c, _):
            pltpu.sync_copy(
                bpix_hbm.at[pl.ds(pl.multiple_of(start + c * CH3, 16), CH3)],
                pixbuf)

            def vec_body(i, _):
                pix = pixbuf[pl.ds(i * 16, 16)]
                pos = c * CH3 + i * 16 + iota
                pt = pos + tag
                valid = (pos < cnt) & (pix < PIX_SENT)
                pixc = pix & (BSZ - 1)

                def converge(carry):
                    wv = plsc.load_gather(idxreg, [pixc])
                    m = (pt > wv) & valid
                    plsc.store_scatter(idxreg, [pixc], pt, mask=m)
                    return jnp.any(m)
                lax.while_loop(lambda t: t, converge, jnp.bool_(True))
                return 0
            lax.fori_loop(0, CH3 // 16, vec_body, 0)
            return 0
        lax.fori_loop(0, nch, chunk_a, 0)

        def chunk_b(c, _):
            pltpu.sync_copy(
                bpix_hbm.at[pl.ds(pl.multiple_of(start + c * CH3, 16), CH3)],
                pixbuf)
            pltpu.sync_copy(
                bint_hbm.at[pl.ds(pl.multiple_of(start + c * CH3, 16), CH3)],
                intbuf)

            def vec_body(i, _):
                pix = pixbuf[pl.ds(i * 16, 16)]
                pos = c * CH3 + i * 16 + iota
                pt = pos + tag
                valid = (pos < cnt) & (pix < PIX_SENT)
                pixc = pix & (BSZ - 1)
                wv = plsc.load_gather(idxreg, [pixc])
                m = (wv == pt) & valid
                plsc.store_scatter(valreg, [pixc], intbuf[pl.ds(i * 16, 16)],
                                   mask=m)
                return 0
            lax.fori_loop(0, CH3 // 16, vec_body, 0)
            return 0
        lax.fori_loop(0, nch, chunk_b, 0)

        pltpu.sync_copy(valreg, out_hbm.at[pl.ds(b * BSZ, BSZ)])
        return 0
    lax.fori_loop(0, 16, bucket_body, 0)


def _sc_scatter(keys, inten_pad):
    mesh = plsc.VectorSubcoreMesh(core_axis_name="c", subcore_axis_name="s")
    params = pltpu.CompilerParams(needs_layout_passes=False,
                                  use_tc_tiling_on_sc=False)

    k1 = pl.kernel(
        _sc_count_body,
        out_type=[jax.ShapeDtypeStruct((NT * NB,), jnp.int32),
                  jax.ShapeDtypeStruct((NT * CNTW,), jnp.int32)],
        mesh=mesh,
        scratch_types=[pltpu.VMEM((CNTW,), jnp.int32),
                       pltpu.VMEM((NB,), jnp.int32),
                       pltpu.VMEM((CH,), jnp.int32)],
        compiler_params=params,
    )
    tb, cnt3 = k1(keys)

    k2 = pl.kernel(
        _sc_place_body,
        out_type=[jax.ShapeDtypeStruct((GROWS, 16), jnp.int32),
                  jax.ShapeDtypeStruct((GROWS, 16), jnp.float32)],
        mesh=mesh,
        scratch_types=[pltpu.VMEM((NT * NB,), jnp.int32),
                       pltpu.VMEM((640,), jnp.int32),
                       pltpu.VMEM((LROWS, 16), jnp.int32),
                       pltpu.VMEM((LROWS, 16), jnp.float32),
                       pltpu.VMEM((2560,), jnp.int32),
                       pltpu.VMEM((2482,), jnp.int32),
                       pltpu.VMEM((512,), jnp.int32),
                       pltpu.VMEM((CH,), jnp.int32),
                       pltpu.VMEM((CH,), jnp.float32),
                       pltpu.VMEM((128,), jnp.int32),
                       pltpu.SemaphoreType.DMA],
        compiler_params=params,
    )
    bpix, bint = k2(keys, inten_pad, tb)
    bpix = bpix.reshape(-1)
    bint = bint.reshape(-1)

    k3 = pl.kernel(
        _sc_emit_body,
        out_type=jax.ShapeDtypeStruct((SENT,), jnp.float32),
        mesh=mesh,
        scratch_types=[pltpu.VMEM((NT * NB,), jnp.int32),
                       pltpu.VMEM((BSZ,), jnp.int32),
                       pltpu.VMEM((BSZ,), jnp.float32),
                       pltpu.VMEM((CH3,), jnp.int32),
                       pltpu.VMEM((CH3,), jnp.float32),
                       pltpu.SMEM((16,), jnp.int32),
                       pltpu.SMEM((16,), jnp.int32)],
        compiler_params=params,
    )
    return k3(bpix, bint, tb)


def kernel(points, cam_params):
    keys, inten_pad = _compute_keys(points, cam_params)
    buf = _sc_scatter(keys, inten_pad)
    front = buf[0:S * S].reshape(S, S)
    back = buf[S * S:2 * S * S].reshape(S, S)
    left = buf[2 * S * S:3 * S * S].reshape(S, S)
    right = buf[3 * S * S:4 * S * S].reshape(S, S)
    return front, back, left, right
